# baseline (speedup 1.0000x reference)
import functools

import jax
import jax.numpy as jnp
from jax import lax
from jax.experimental import pallas as pl
from jax.experimental.pallas import tpu as pltpu
from jax.experimental.pallas import tpu_sc as plsc

N0, N1, N2 = 10000, 4000, 1024
E1, E2 = 320000, 128000
F_IN, H, C = 128, 64, 10

_NW = 32
_K = 128


def _make_gat_edges(n_src, n_dst, nchunk, real_groups, local_acc=False):
    full_rows = real_groups // 8
    rem = real_groups % 8
    rows_per_tile = n_dst // 16

    mesh = plsc.VectorSubcoreMesh(core_axis_name="c", subcore_axis_name="s")

    @functools.partial(
        pl.kernel,
        out_type=(
            jax.ShapeDtypeStruct((2, n_dst, H), jnp.float32),
            jax.ShapeDtypeStruct((2, n_dst), jnp.float32),
        ),
        mesh=mesh,
        compiler_params=pltpu.CompilerParams(
            needs_layout_passes=False, use_tc_tiling_on_sc=False),
        scratch_types=[
            pltpu.VMEM((n_src,), jnp.float32),
            pltpu.VMEM((n_dst,), jnp.float32),
            pltpu.VMEM((nchunk, _K), jnp.int32),
            pltpu.VMEM((nchunk, _K), jnp.int32),
            pltpu.VMEM((nchunk, _K), jnp.float32),
            [pltpu.VMEM((_K, H), jnp.float32) for _ in range(4)],
            pltpu.VMEM((16,), jnp.float32),
            pltpu.VMEM_SHARED((n_dst, H), jnp.float32),
            pltpu.VMEM_SHARED((n_dst,), jnp.float32),
            [pltpu.SemaphoreType.DMA for _ in range(4)],
            [pltpu.SemaphoreType.DMA for _ in range(4)],
            pltpu.SemaphoreType.DMA,
            (pltpu.VMEM((n_dst, H), jnp.float32) if local_acc
             else pltpu.VMEM((16,), jnp.float32)),
            pltpu.VMEM((max(n_dst // 128, 1), 128), jnp.int32),
        ],
    )
    def kern(hs_hbm, as_hbm, ad_hbm, src_hbm, dst_hbm, m_hbm, zacc_hbm, zs_hbm,
             iota_hbm, acc_out, s_out,
             as_v, ad_v, src_v, dst_v, ex_v, rows, m_v,
             acc_sh, s_sh, gsem, ssem, xsem, accl_v, iota_v):
        cid = lax.axis_index("c")
        sid = lax.axis_index("s")
        wid = cid * 16 + sid

        pltpu.sync_copy(as_hbm, as_v)
        pltpu.sync_copy(ad_hbm, ad_v)
        pltpu.sync_copy(src_hbm.at[wid], src_v)
        pltpu.sync_copy(dst_hbm.at[wid], dst_v)
        pltpu.sync_copy(m_hbm, m_v)

        @pl.when(sid == 0)
        def _():
            pltpu.sync_copy(zacc_hbm, acc_sh)
            pltpu.sync_copy(zs_hbm, s_sh)

        if local_acc:
            pltpu.sync_copy(iota_hbm, iota_v)
            zero16 = jnp.zeros((16,), jnp.float32)

            def zrow(r, c0):
                for q in range(H // 16):
                    accl_v[r, pl.ds(q * 16, 16)] = zero16
                return c0

            lax.fori_loop(0, n_dst, zrow, 0)

        mvec = m_v[...]

        def group_body(r, c):
            si = src_v[r, pl.ds(c, 16)]
            di = dst_v[r, pl.ds(c, 16)]
            e = plsc.load_gather(as_v, [si]) + plsc.load_gather(ad_v, [di])
            e = jnp.where(e < 0.0, e * 0.2, e)
            ex_v[r, pl.ds(c, 16)] = jnp.exp(e - mvec)

        def row_body(r, carry):
            for g in range(8):
                group_body(r, g * 16)
            return carry

        lax.fori_loop(0, full_rows, row_body, 0)
        zero16 = jnp.zeros((16,), jnp.float32)
        for g in range(8):
            if g < rem:
                group_body(full_rows, g * 16)
            elif rem:
                ex_v[full_rows, pl.ds(g * 16, 16)] = zero16

        plsc.subcore_barrier()

        def scale(rows_v, j):
            def edge_group(g, c2):
                exg = ex_v[j, pl.ds(g * 16, 16)]
                for kk in range(16):
                    w = jnp.full((16,), exg[kk])
                    k = g * 16 + kk
                    for q in range(H // 16):
                        sl = pl.ds(q * 16, 16)
                        rows_v[k, sl] = rows_v[k, sl] * w
                return c2

            lax.fori_loop(0, _K // 16, edge_group, 0)

        def fire_g(rows_v, sem, j):
            pltpu.async_copy(hs_hbm.at[src_v.at[j]], rows_v, sem)

        def drain_g(rows_v, sem, j):
            pltpu.make_async_copy(hs_hbm.at[src_v.at[j]], rows_v, sem).wait()

        def fire_sc(rows_v, sem, j):
            pltpu.async_copy(rows_v, acc_sh.at[dst_v.at[j]], sem, add=True)
            pltpu.async_copy(ex_v.at[j], s_sh.at[dst_v.at[j]], xsem, add=True)

        def drain_sc(rows_v, sem, j):
            pltpu.make_async_copy(rows_v, acc_sh.at[dst_v.at[j]], sem).wait()

        def accum(rows_v, j):
            def edge_group(g, c2):
                exg = ex_v[j, pl.ds(g * 16, 16)]
                dstg = dst_v[j, pl.ds(g * 16, 16)]
                for kk in range(16):
                    w = jnp.full((16,), exg[kk])
                    d = dstg[kk]
                    k = g * 16 + kk
                    for q in range(H // 16):
                        sl = pl.ds(q * 16, 16)
                        accl_v[d, sl] = accl_v[d, sl] + rows_v[k, sl] * w
                return c2

            lax.fori_loop(0, _K // 16, edge_group, 0)

        for b in range(3):
            fire_g(rows[b], gsem[b], b)

        def step(b, j, fire_next=True):
            drain_g(rows[b], gsem[b], j)
            if local_acc:
                accum(rows[b], j)
                pltpu.async_copy(ex_v.at[j], s_sh.at[dst_v.at[j]], xsem,
                                 add=True)
            else:
                scale(rows[b], j)
                fire_sc(rows[b], ssem[b], j)
            if not fire_next:
                return
            t = (b + 3) % 4
            jn = j + 3

            if not local_acc:
                @pl.when((jn < nchunk) & (j >= 1))
                def _():
                    drain_sc(rows[t], ssem[t], 0)

            @pl.when(jn < nchunk)
            def _():
                fire_g(rows[t], gsem[t], jn)

        def quad_body(q, carry):
            for b in range(4):
                step(b, q * 4 + b)
            return carry

        nquads = nchunk // 4
        lax.fori_loop(0, nquads, quad_body, 0)
        for jt in range(nquads * 4, nchunk):
            step(jt % 4, jt, fire_next=False)
        if local_acc:
            for r in range(n_dst // 128):
                pltpu.sync_copy(accl_v.at[pl.ds(r * 128, 128)],
                                acc_sh.at[iota_v.at[r]], add=True)
        else:
            for b in range(4):
                drain_sc(rows[b], ssem[b], 0)

        def drain_x(j, carry):
            pltpu.make_async_copy(
                ex_v.at[j], s_sh.at[dst_v.at[j]], xsem).wait()
            return carry

        lax.fori_loop(0, nchunk, drain_x, 0)

        plsc.subcore_barrier()
        slab = (rows_per_tile + 7) // 8 * 8
        last = n_dst - 15 * slab

        @pl.when(sid < 15)
        def _():
            r0 = sid * slab
            pltpu.sync_copy(acc_sh.at[pl.ds(r0, slab)],
                            acc_out.at[cid].at[pl.ds(r0, slab)])

        @pl.when(sid == 15)
        def _():
            pltpu.sync_copy(acc_sh.at[pl.ds(15 * slab, last)],
                            acc_out.at[cid].at[pl.ds(15 * slab, last)])

        @pl.when(sid == 0)
        def _():
            pltpu.sync_copy(s_sh, s_out.at[cid])

    return kern


_gat_edges_1 = _make_gat_edges(N0, N1, 79, 625)
_gat_edges_2 = _make_gat_edges(N1, N2, 32, 250)


def _tc_proj_body(x_ref, w_ref, avs_ref, avd_ref,
                  hs_ref, as_ref, ad_ref, msa_ref, mda_ref):
    i = pl.program_id(0)
    hs = jnp.dot(x_ref[...], w_ref[...], preferred_element_type=jnp.float32)
    hs_ref[...] = hs
    a_s = jnp.dot(hs, avs_ref[...], preferred_element_type=jnp.float32)
    a_d = jnp.dot(hs, avd_ref[...], preferred_element_type=jnp.float32)
    as_ref[...] = a_s
    ad_ref[...] = a_d
    neg = jnp.full((1, 1), -jnp.inf, jnp.float32)
    prev_s = jnp.where(i == 0, neg, msa_ref[...])
    prev_d = jnp.where(i == 0, neg, mda_ref[...])
    msa_ref[...] = jnp.maximum(prev_s, jnp.full((1, 1), jnp.max(a_s)))
    mda_ref[...] = jnp.maximum(prev_d, jnp.full((1, 1), jnp.max(a_d)))


def _tc_proj(x, w, av_s, av_d, blk):
    n = x.shape[0]
    f = x.shape[1]
    grid = n // blk
    return pl.pallas_call(
        _tc_proj_body,
        grid=(grid,),
        in_specs=[
            pl.BlockSpec((blk, f), lambda i: (i, 0)),
            pl.BlockSpec((f, H), lambda i: (0, 0)),
            pl.BlockSpec((H, 1), lambda i: (0, 0)),
            pl.BlockSpec((H, 1), lambda i: (0, 0)),
        ],
        out_specs=[
            pl.BlockSpec((blk, H), lambda i: (i, 0)),
            pl.BlockSpec((blk, 1), lambda i: (i, 0)),
            pl.BlockSpec((blk, 1), lambda i: (i, 0)),
            pl.BlockSpec((1, 1), lambda i: (0, 0)),
            pl.BlockSpec((1, 1), lambda i: (0, 0)),
        ],
        out_shape=[
            jax.ShapeDtypeStruct((n, H), jnp.float32),
            jax.ShapeDtypeStruct((n, 1), jnp.float32),
            jax.ShapeDtypeStruct((n, 1), jnp.float32),
            jax.ShapeDtypeStruct((1, 1), jnp.float32),
            jax.ShapeDtypeStruct((1, 1), jnp.float32),
        ],
    )(x, w, av_s, av_d)


def _tc_norm_proj_body(acc_ref, sp_ref, b_ref, w_ref, avs_ref, avd_ref,
                       hs_ref, as_ref, ad_ref, msa_ref, mda_ref):
    i = pl.program_id(0)
    s = sp_ref[...][:, 0:1] + sp_ref[...][:, 1:2]
    h = (acc_ref[0] + acc_ref[1]) / (s + 1e-16) + b_ref[...]
    h = jnp.maximum(h, 0.0)
    hs = jnp.dot(h, w_ref[...], preferred_element_type=jnp.float32)
    hs_ref[...] = hs
    a_s = jnp.dot(hs, avs_ref[...], preferred_element_type=jnp.float32)
    a_d = jnp.dot(hs, avd_ref[...], preferred_element_type=jnp.float32)
    as_ref[...] = a_s
    ad_ref[...] = a_d
    neg = jnp.full((1, 1), -jnp.inf, jnp.float32)
    prev_s = jnp.where(i == 0, neg, msa_ref[...])
    prev_d = jnp.where(i == 0, neg, mda_ref[...])
    msa_ref[...] = jnp.maximum(prev_s, jnp.full((1, 1), jnp.max(a_s)))
    mda_ref[...] = jnp.maximum(prev_d, jnp.full((1, 1), jnp.max(a_d)))


def _tc_norm_proj(acc_parts, s_pair, b, w, av_s, av_d, blk):
    n = acc_parts.shape[1]
    grid = n // blk
    return pl.pallas_call(
        _tc_norm_proj_body,
        grid=(grid,),
        in_specs=[
            pl.BlockSpec((2, blk, H), lambda i: (0, i, 0)),
            pl.BlockSpec((blk, 2), lambda i: (i, 0)),
            pl.BlockSpec((1, H), lambda i: (0, 0)),
            pl.BlockSpec((H, H), lambda i: (0, 0)),
            pl.BlockSpec((H, 1), lambda i: (0, 0)),
            pl.BlockSpec((H, 1), lambda i: (0, 0)),
        ],
        out_specs=[
            pl.BlockSpec((blk, H), lambda i: (i, 0)),
            pl.BlockSpec((blk, 1), lambda i: (i, 0)),
            pl.BlockSpec((blk, 1), lambda i: (i, 0)),
            pl.BlockSpec((1, 1), lambda i: (0, 0)),
            pl.BlockSpec((1, 1), lambda i: (0, 0)),
        ],
        out_shape=[
            jax.ShapeDtypeStruct((n, H), jnp.float32),
            jax.ShapeDtypeStruct((n, 1), jnp.float32),
            jax.ShapeDtypeStruct((n, 1), jnp.float32),
            jax.ShapeDtypeStruct((1, 1), jnp.float32),
            jax.ShapeDtypeStruct((1, 1), jnp.float32),
        ],
    )(acc_parts, s_pair, b, w, av_s, av_d)


def _tc_head_body(acc_ref, sp_ref, b_ref, w_ref, bl_ref, out_ref):
    s = sp_ref[...][:, 0:1] + sp_ref[...][:, 1:2]
    h = (acc_ref[0] + acc_ref[1]) / (s + 1e-16) + b_ref[...]
    o = jnp.dot(h, w_ref[...], preferred_element_type=jnp.float32) + bl_ref[...]
    m = jnp.max(o, axis=1, keepdims=True)
    lse = jnp.log(jnp.sum(jnp.exp(o - m), axis=1, keepdims=True)) + m
    out_ref[...] = o - lse


def _tc_head(acc_parts, s_pair, b, wl, bl):
    n = acc_parts.shape[1]
    return pl.pallas_call(
        _tc_head_body,
        out_shape=jax.ShapeDtypeStruct((n, C), jnp.float32),
    )(acc_parts, s_pair, b, wl, bl)


def _pack_edges(idx, nchunk):
    per_tile = idx.shape[0] // _NW
    pad = nchunk * _K - per_tile
    a = idx.reshape(_NW, per_tile)
    a = jnp.pad(a, ((0, 0), (0, pad)))
    return a.reshape(_NW, nchunk, _K)


def kernel(x, src1, dst1, src2, dst2, W1, a1s, a1d, b1, W2, a2s, a2d, b2, Wl, bl):
    f32 = jnp.float32
    zacc1 = jnp.zeros((N1, H), f32)
    zs1 = jnp.zeros((N1,), f32)
    zacc2 = jnp.zeros((N2, H), f32)
    zs2 = jnp.zeros((N2,), f32)

    hs1, as1, ad1, msa1, mda1 = _tc_proj(
        x, W1, a1s.reshape(H, 1), a1d.reshape(H, 1), 1000)
    m1 = jax.nn.leaky_relu(msa1[0, 0] + mda1[0, 0], 0.2)
    m1v = jnp.broadcast_to(m1, (16,))

    sp1 = _pack_edges(src1, 79)
    dp1 = _pack_edges(dst1, 79)
    iota1 = jnp.zeros((N1 // 128, 128), jnp.int32)
    acc1, s1 = _gat_edges_1(
        hs1, as1.reshape(N0), ad1.reshape(N0)[:N1], sp1, dp1, m1v,
        zacc1, zs1, iota1)

    hs2, as2, ad2, msa2, mda2 = _tc_norm_proj(
        acc1, s1.T, b1.reshape(1, H), W2,
        a2s.reshape(H, 1), a2d.reshape(H, 1), 1000)
    m2 = jax.nn.leaky_relu(msa2[0, 0] + mda2[0, 0], 0.2)
    m2v = jnp.broadcast_to(m2, (16,))

    sp2 = _pack_edges(src2, 32)
    dp2 = _pack_edges(dst2, 32)
    iota2 = jnp.arange(N2, dtype=jnp.int32).reshape(N2 // 128, 128)
    acc2, s2 = _gat_edges_2(
        hs2, as2.reshape(N1), ad2.reshape(N1)[:N2], sp2, dp2, m2v,
        zacc2, zs2, iota2)

    return _tc_head(acc2, s2.T, b2.reshape(1, H), Wl, bl)


# speedup vs baseline: 1.0013x; 1.0013x over previous
import functools

import jax
import jax.numpy as jnp
from jax import lax
from jax.experimental import pallas as pl
from jax.experimental.pallas import tpu as pltpu
from jax.experimental.pallas import tpu_sc as plsc

N0, N1, N2 = 10000, 4000, 1024
E1, E2 = 320000, 128000
F_IN, H, C = 128, 64, 10

_NW = 32
_K = 128


def _make_gat_edges(n_src, n_dst, nchunk, real_groups):
    full_rows = real_groups // 8
    rem = real_groups % 8
    rows_per_tile = n_dst // 16

    mesh = plsc.VectorSubcoreMesh(core_axis_name="c", subcore_axis_name="s")

    @functools.partial(
        pl.kernel,
        out_type=(
            jax.ShapeDtypeStruct((2, n_dst, H), jnp.float32),
            jax.ShapeDtypeStruct((2, n_dst), jnp.float32),
        ),
        mesh=mesh,
        compiler_params=pltpu.CompilerParams(
            needs_layout_passes=False, use_tc_tiling_on_sc=False),
        scratch_types=[
            pltpu.VMEM((n_src,), jnp.float32),
            pltpu.VMEM((n_dst,), jnp.float32),
            pltpu.VMEM((nchunk, _K), jnp.int32),
            pltpu.VMEM((nchunk, _K), jnp.int32),
            pltpu.VMEM((nchunk, _K), jnp.float32),
            [pltpu.VMEM((_K, H), jnp.float32) for _ in range(4)],
            pltpu.VMEM((16,), jnp.float32),
            pltpu.VMEM_SHARED((n_dst, H), jnp.float32),
            pltpu.VMEM_SHARED((n_dst,), jnp.float32),
            [pltpu.SemaphoreType.DMA for _ in range(4)],
            [pltpu.SemaphoreType.DMA for _ in range(4)],
            pltpu.SemaphoreType.DMA,
        ],
    )
    def kern(hs_hbm, as_hbm, ad_hbm, src_hbm, dst_hbm, m_hbm, zacc_hbm, zs_hbm,
             acc_out, s_out,
             as_v, ad_v, src_v, dst_v, ex_v, rows, m_v,
             acc_sh, s_sh, gsem, ssem, xsem):
        cid = lax.axis_index("c")
        sid = lax.axis_index("s")
        wid = cid * 16 + sid

        pltpu.sync_copy(as_hbm, as_v)
        pltpu.sync_copy(ad_hbm, ad_v)
        pltpu.sync_copy(src_hbm.at[wid], src_v)
        pltpu.sync_copy(dst_hbm.at[wid], dst_v)
        pltpu.sync_copy(m_hbm, m_v)

        @pl.when(sid == 0)
        def _():
            pltpu.sync_copy(zacc_hbm, acc_sh)
            pltpu.sync_copy(zs_hbm, s_sh)

        mvec = m_v[...]

        def group_body(r, c):
            si = src_v[r, pl.ds(c, 16)]
            di = dst_v[r, pl.ds(c, 16)]
            e = plsc.load_gather(as_v, [si]) + plsc.load_gather(ad_v, [di])
            e = jnp.where(e < 0.0, e * 0.2, e)
            ex_v[r, pl.ds(c, 16)] = jnp.exp(e - mvec)

        def row_body(r, carry):
            for g in range(8):
                group_body(r, g * 16)
            return carry

        lax.fori_loop(0, full_rows, row_body, 0)
        zero16 = jnp.zeros((16,), jnp.float32)
        for g in range(8):
            if g < rem:
                group_body(full_rows, g * 16)
            elif rem:
                ex_v[full_rows, pl.ds(g * 16, 16)] = zero16

        plsc.subcore_barrier()

        def scale(rows_v, j):
            def edge_group(g, c2):
                exg = ex_v[j, pl.ds(g * 16, 16)]
                for kk in range(16):
                    w = jnp.full((16,), exg[kk])
                    k = g * 16 + kk
                    for q in range(H // 16):
                        sl = pl.ds(q * 16, 16)
                        rows_v[k, sl] = rows_v[k, sl] * w
                return c2

            lax.fori_loop(0, _K // 16, edge_group, 0)

        def fire_g(rows_v, sem, j):
            pltpu.async_copy(hs_hbm.at[src_v.at[j]], rows_v, sem)

        def drain_g(rows_v, sem, j):
            pltpu.make_async_copy(hs_hbm.at[src_v.at[j]], rows_v, sem).wait()

        def fire_sc(rows_v, sem, j):
            pltpu.async_copy(rows_v, acc_sh.at[dst_v.at[j]], sem, add=True)
            pltpu.async_copy(ex_v.at[j], s_sh.at[dst_v.at[j]], xsem, add=True)

        def drain_sc(rows_v, sem, j):
            pltpu.make_async_copy(rows_v, acc_sh.at[dst_v.at[j]], sem).wait()

        for b in range(3):
            fire_g(rows[b], gsem[b], b)

        def step(b, j, fire_next=True):
            drain_g(rows[b], gsem[b], j)
            scale(rows[b], j)
            fire_sc(rows[b], ssem[b], j)
            if not fire_next:
                return
            t = (b + 3) % 4
            jn = j + 3

            @pl.when((jn < nchunk) & (j >= 1))
            def _():
                drain_sc(rows[t], ssem[t], 0)

            @pl.when(jn < nchunk)
            def _():
                fire_g(rows[t], gsem[t], jn)

        def quad_body(q, carry):
            for b in range(4):
                step(b, q * 4 + b)
            return carry

        nquads = nchunk // 4
        lax.fori_loop(0, nquads, quad_body, 0)
        for jt in range(nquads * 4, nchunk):
            step(jt % 4, jt, fire_next=False)
        for b in range(4):
            drain_sc(rows[b], ssem[b], 0)

        def drain_x(j, carry):
            pltpu.make_async_copy(
                ex_v.at[j], s_sh.at[dst_v.at[j]], xsem).wait()
            return carry

        lax.fori_loop(0, nchunk, drain_x, 0)

        plsc.subcore_barrier()
        slab = (rows_per_tile + 7) // 8 * 8
        last = n_dst - 15 * slab

        @pl.when(sid < 15)
        def _():
            r0 = sid * slab
            pltpu.sync_copy(acc_sh.at[pl.ds(r0, slab)],
                            acc_out.at[cid].at[pl.ds(r0, slab)])

        @pl.when(sid == 15)
        def _():
            pltpu.sync_copy(acc_sh.at[pl.ds(15 * slab, last)],
                            acc_out.at[cid].at[pl.ds(15 * slab, last)])

        @pl.when(sid == 0)
        def _():
            pltpu.sync_copy(s_sh, s_out.at[cid])

    return kern


_gat_edges_1 = _make_gat_edges(N0, N1, 79, 625)
_gat_edges_2 = _make_gat_edges(N1, N2, 32, 250)


def _tc_proj_body(x_ref, w_ref, avs_ref, avd_ref,
                  hs_ref, as_ref, ad_ref, msa_ref, mda_ref):
    i = pl.program_id(0)
    hs = jnp.dot(x_ref[...], w_ref[...], preferred_element_type=jnp.float32)
    hs_ref[...] = hs
    a_s = jnp.dot(hs, avs_ref[...], preferred_element_type=jnp.float32)
    a_d = jnp.dot(hs, avd_ref[...], preferred_element_type=jnp.float32)
    as_ref[...] = a_s
    ad_ref[...] = a_d
    neg = jnp.full((1, 1), -jnp.inf, jnp.float32)
    prev_s = jnp.where(i == 0, neg, msa_ref[...])
    prev_d = jnp.where(i == 0, neg, mda_ref[...])
    msa_ref[...] = jnp.maximum(prev_s, jnp.full((1, 1), jnp.max(a_s)))
    mda_ref[...] = jnp.maximum(prev_d, jnp.full((1, 1), jnp.max(a_d)))


def _tc_proj(x, w, av_s, av_d, blk):
    n = x.shape[0]
    f = x.shape[1]
    grid = n // blk
    return pl.pallas_call(
        _tc_proj_body,
        grid=(grid,),
        in_specs=[
            pl.BlockSpec((blk, f), lambda i: (i, 0)),
            pl.BlockSpec((f, H), lambda i: (0, 0)),
            pl.BlockSpec((H, 1), lambda i: (0, 0)),
            pl.BlockSpec((H, 1), lambda i: (0, 0)),
        ],
        out_specs=[
            pl.BlockSpec((blk, H), lambda i: (i, 0)),
            pl.BlockSpec((blk, 1), lambda i: (i, 0)),
            pl.BlockSpec((blk, 1), lambda i: (i, 0)),
            pl.BlockSpec((1, 1), lambda i: (0, 0)),
            pl.BlockSpec((1, 1), lambda i: (0, 0)),
        ],
        out_shape=[
            jax.ShapeDtypeStruct((n, H), jnp.float32),
            jax.ShapeDtypeStruct((n, 1), jnp.float32),
            jax.ShapeDtypeStruct((n, 1), jnp.float32),
            jax.ShapeDtypeStruct((1, 1), jnp.float32),
            jax.ShapeDtypeStruct((1, 1), jnp.float32),
        ],
    )(x, w, av_s, av_d)


def _tc_norm_proj_body(acc_ref, sp_ref, b_ref, w_ref, avs_ref, avd_ref,
                       hs_ref, as_ref, ad_ref, msa_ref, mda_ref):
    i = pl.program_id(0)
    s = sp_ref[...][:, 0:1] + sp_ref[...][:, 1:2]
    h = (acc_ref[0] + acc_ref[1]) / (s + 1e-16) + b_ref[...]
    h = jnp.maximum(h, 0.0)
    hs = jnp.dot(h, w_ref[...], preferred_element_type=jnp.float32)
    hs_ref[...] = hs
    a_s = jnp.dot(hs, avs_ref[...], preferred_element_type=jnp.float32)
    a_d = jnp.dot(hs, avd_ref[...], preferred_element_type=jnp.float32)
    as_ref[...] = a_s
    ad_ref[...] = a_d
    neg = jnp.full((1, 1), -jnp.inf, jnp.float32)
    prev_s = jnp.where(i == 0, neg, msa_ref[...])
    prev_d = jnp.where(i == 0, neg, mda_ref[...])
    msa_ref[...] = jnp.maximum(prev_s, jnp.full((1, 1), jnp.max(a_s)))
    mda_ref[...] = jnp.maximum(prev_d, jnp.full((1, 1), jnp.max(a_d)))


def _tc_norm_proj(acc_parts, s_pair, b, w, av_s, av_d, blk):
    n = acc_parts.shape[1]
    grid = n // blk
    return pl.pallas_call(
        _tc_norm_proj_body,
        grid=(grid,),
        in_specs=[
            pl.BlockSpec((2, blk, H), lambda i: (0, i, 0)),
            pl.BlockSpec((blk, 2), lambda i: (i, 0)),
            pl.BlockSpec((1, H), lambda i: (0, 0)),
            pl.BlockSpec((H, H), lambda i: (0, 0)),
            pl.BlockSpec((H, 1), lambda i: (0, 0)),
            pl.BlockSpec((H, 1), lambda i: (0, 0)),
        ],
        out_specs=[
            pl.BlockSpec((blk, H), lambda i: (i, 0)),
            pl.BlockSpec((blk, 1), lambda i: (i, 0)),
            pl.BlockSpec((blk, 1), lambda i: (i, 0)),
            pl.BlockSpec((1, 1), lambda i: (0, 0)),
            pl.BlockSpec((1, 1), lambda i: (0, 0)),
        ],
        out_shape=[
            jax.ShapeDtypeStruct((n, H), jnp.float32),
            jax.ShapeDtypeStruct((n, 1), jnp.float32),
            jax.ShapeDtypeStruct((n, 1), jnp.float32),
            jax.ShapeDtypeStruct((1, 1), jnp.float32),
            jax.ShapeDtypeStruct((1, 1), jnp.float32),
        ],
    )(acc_parts, s_pair, b, w, av_s, av_d)


def _tc_head_body(acc_ref, sp_ref, b_ref, w_ref, bl_ref, out_ref):
    s = sp_ref[...][:, 0:1] + sp_ref[...][:, 1:2]
    h = (acc_ref[0] + acc_ref[1]) / (s + 1e-16) + b_ref[...]
    o = jnp.dot(h, w_ref[...], preferred_element_type=jnp.float32) + bl_ref[...]
    m = jnp.max(o, axis=1, keepdims=True)
    lse = jnp.log(jnp.sum(jnp.exp(o - m), axis=1, keepdims=True)) + m
    out_ref[...] = o - lse


def _tc_head(acc_parts, s_pair, b, wl, bl):
    n = acc_parts.shape[1]
    return pl.pallas_call(
        _tc_head_body,
        out_shape=jax.ShapeDtypeStruct((n, C), jnp.float32),
    )(acc_parts, s_pair, b, wl, bl)


def _pack_edges(idx, nchunk):
    per_tile = idx.shape[0] // _NW
    pad = nchunk * _K - per_tile
    a = idx.reshape(_NW, per_tile)
    a = jnp.pad(a, ((0, 0), (0, pad)))
    return a.reshape(_NW, nchunk, _K)


def kernel(x, src1, dst1, src2, dst2, W1, a1s, a1d, b1, W2, a2s, a2d, b2, Wl, bl):
    f32 = jnp.float32
    zacc1 = jnp.zeros((N1, H), f32)
    zs1 = jnp.zeros((N1,), f32)
    zacc2 = jnp.zeros((N2, H), f32)
    zs2 = jnp.zeros((N2,), f32)

    hs1, as1, ad1, msa1, mda1 = _tc_proj(
        x, W1, a1s.reshape(H, 1), a1d.reshape(H, 1), 1000)
    m1 = jax.nn.leaky_relu(msa1[0, 0] + mda1[0, 0], 0.2)
    m1v = jnp.broadcast_to(m1, (16,))

    sp1 = _pack_edges(src1, 79)
    dp1 = _pack_edges(dst1, 79)
    acc1, s1 = _gat_edges_1(
        hs1, as1.reshape(N0), ad1.reshape(N0)[:N1], sp1, dp1, m1v,
        zacc1, zs1)

    hs2, as2, ad2, msa2, mda2 = _tc_norm_proj(
        acc1, s1.T, b1.reshape(1, H), W2,
        a2s.reshape(H, 1), a2d.reshape(H, 1), 1000)
    m2 = jax.nn.leaky_relu(msa2[0, 0] + mda2[0, 0], 0.2)
    m2v = jnp.broadcast_to(m2, (16,))

    sp2 = _pack_edges(src2, 32)
    dp2 = _pack_edges(dst2, 32)
    acc2, s2 = _gat_edges_2(
        hs2, as2.reshape(N1), ad2.reshape(N1)[:N2], sp2, dp2, m2v,
        zacc2, zs2)

    return _tc_head(acc2, s2.T, b2.reshape(1, H), Wl, bl)
